# baseline (speedup 1.0000x reference)
import jax
import jax.numpy as jnp
from jax import lax
from jax.experimental import pallas as pl

B, S, DM, QIN = 1, 2048, 1024, 1024
H, D, TOPK = 16, 64, 512
R = 128
SCALE = D ** -0.5
WSCALE = H ** -0.5


def _substep(v, i, desc, d, final_desc):
    n = v.shape[0]
    if d >= 1:
        o = n // (2 * d)
        v4 = v.reshape(o, 2, d, R)
        i4 = i.reshape(o, 2, d, R)
        va, vb = v4[:, 0], v4[:, 1]
        ia, ib = i4[:, 0], i4[:, 1]
        r = (va > vb) | ((va == vb) & (ia < ib))
        if final_desc:
            swap = ~r
        else:
            desc4 = desc.reshape(o, 2, d, 1)[:, 0]
            swap = r ^ desc4
        na = jnp.where(swap, vb, va)
        nb = jnp.where(swap, va, vb)
        nia = jnp.where(swap, ib, ia)
        nib = jnp.where(swap, ia, ib)
        v = jnp.concatenate([na[:, None], nb[:, None]], axis=1).reshape(n, R)
        i = jnp.concatenate([nia[:, None], nib[:, None]], axis=1).reshape(n, R)
    else:
        iota_col = lax.broadcasted_iota(jnp.int32, (n, 1), 0)
        t = ((iota_col // d) & 1) == 1
        vm = jnp.roll(v, -d, axis=0)
        vp = jnp.roll(v, d, axis=0)
        pv = jnp.where(t, vp, vm)
        im = jnp.roll(i, -d, axis=0)
        ip = jnp.roll(i, d, axis=0)
        pi = jnp.where(t, ip, im)
        r = (v > pv) | ((v == pv) & (i < pi))
        if final_desc:
            keep_first = ~t
        else:
            keep_first = desc ^ t
        sel = r == keep_first
        v = jnp.where(sel, v, pv)
        i = jnp.where(sel, i, pi)
    return v, i


def _sort_chunk(v, i, chunk_desc):
    iota_c = lax.broadcasted_iota(jnp.int32, (512, 1), 0)
    ones = jnp.ones_like(iota_c) > 0
    for k in range(1, 10):
        if k <= 8:
            desc = ((iota_c >> (8 - k)) & 1) == 0
        else:
            desc = ones if chunk_desc else ~ones
        for a in range(k - 1, -1, -1):
            v, i = _substep(v, i, desc, 1 << (8 - a), False)
    return v, i


def _merge_chunk(v, i, chunk_desc):
    for a in range(8, -1, -1):
        if chunk_desc:
            v, i = _substep(v, i, None, 1 << (8 - a), True)
        else:
            iota_c = lax.broadcasted_iota(jnp.int32, (512, 1), 0)
            desc = iota_c < 0
            v, i = _substep(v, i, desc, 1 << (8 - a), False)
    return v, i


def _bitonic_topk_idx(v):
    n = v.shape[0]
    i = lax.broadcasted_iota(jnp.int32, (n, R), 0)
    cs = []
    for c in range(4):
        vc = v[c * 512:(c + 1) * 512, :]
        ic = i[c * 512:(c + 1) * 512, :]
        cs.append(_sort_chunk(vc, ic, chunk_desc=(c % 2 == 0)))
    halves = []
    for p in range(2):
        (va, ia), (vb, ib) = cs[2 * p], cs[2 * p + 1]
        r = (va > vb) | ((va == vb) & (ia < ib))
        vw = jnp.where(r, va, vb)
        iw = jnp.where(r, ia, ib)
        halves.append(_merge_chunk(vw, iw, chunk_desc=(p == 0)))
    (va, ia), (vb, ib) = halves
    r = (va > vb) | ((va == vb) & (ia < ib))
    v = jnp.where(r, va, vb)
    i = jnp.where(r, ia, ib)
    v, i = _merge_chunk(v, i, chunk_desc=True)
    rank = lax.broadcasted_iota(jnp.int32, (TOPK, 1), 0)
    rev = jnp.zeros_like(rank)
    for b in range(9):
        rev = rev | (((rank >> b) & 1) << (8 - b))
    col = lax.broadcasted_iota(jnp.int32, (TOPK, TOPK), 1)
    onehot = (col == rev).astype(jnp.float32)
    out = jnp.dot(onehot, i.astype(jnp.float32),
                  preferred_element_type=jnp.float32)
    return out.astype(jnp.int32)


def _fused_kernel(qT_in_ref, wq_ref, k_ref, wT_ref, out_ref):
    qT = jnp.dot(wq_ref[...], qT_in_ref[...], preferred_element_type=jnp.float32)
    k = k_ref[...]
    wT = wT_ref[...]
    acc = jnp.zeros((S, R), dtype=jnp.float32)
    for h in range(H):
        sh = jnp.dot(k, qT[h * D:(h + 1) * D, :], preferred_element_type=jnp.float32)
        acc = acc + sh * (wT[h:h + 1, :] * SCALE)
    out_ref[...] = _bitonic_topk_idx(acc)


def _layernorm_host(v, gamma, beta, eps=1e-5):
    mu = jnp.mean(v, axis=-1, keepdims=True)
    var = jnp.var(v, axis=-1, keepdims=True)
    return (v - mu) / jnp.sqrt(var + eps) * gamma + beta


def kernel(x, q_input, Wq, Wk, gamma, beta, Ww):
    x2 = x.reshape(S, DM)
    qT_in = q_input.reshape(S, QIN).T
    k = _layernorm_host(x2 @ Wk.T, gamma, beta)
    wT = ((x2 @ Ww.T) * WSCALE).T

    idxT = pl.pallas_call(
        _fused_kernel,
        grid=(S // R,),
        in_specs=[
            pl.BlockSpec((QIN, R), lambda i: (0, i)),
            pl.BlockSpec((H * D, QIN), lambda i: (0, 0)),
            pl.BlockSpec((S, D), lambda i: (0, 0)),
            pl.BlockSpec((H, R), lambda i: (0, i)),
        ],
        out_specs=pl.BlockSpec((TOPK, R), lambda i: (0, i)),
        out_shape=jax.ShapeDtypeStruct((TOPK, S), jnp.int32),
    )(qT_in, Wq, k, wT)

    return idxT.T.reshape(B, S, TOPK)


# speedup vs baseline: 3.6550x; 3.6550x over previous
import jax
import jax.numpy as jnp
from jax import lax
from jax.experimental import pallas as pl
from jax.experimental.pallas import tpu as pltpu

B, S, DM, QIN = 1, 2048, 1024, 1024
H, D, TOPK = 16, 64, 512
R = 128
SCALE = D ** -0.5
WSCALE = H ** -0.5


def _substep(v, i, desc, d, final_desc):
    n = v.shape[0]
    if d >= 8:
        o = n // (2 * d)
        v4 = v.reshape(o, 2, d, R)
        i4 = i.reshape(o, 2, d, R)
        va, vb = v4[:, 0], v4[:, 1]
        ia, ib = i4[:, 0], i4[:, 1]
        r = (va > vb) | ((va == vb) & (ia < ib))
        if final_desc:
            swap = ~r
        else:
            desc4 = desc.reshape(o, 2, d, 1)[:, 0]
            swap = r ^ desc4
        na = jnp.where(swap, vb, va)
        nb = jnp.where(swap, va, vb)
        nia = jnp.where(swap, ib, ia)
        nib = jnp.where(swap, ia, ib)
        v = jnp.concatenate([na[:, None], nb[:, None]], axis=1).reshape(n, R)
        i = jnp.concatenate([nia[:, None], nib[:, None]], axis=1).reshape(n, R)
    else:
        iota_col = lax.broadcasted_iota(jnp.int32, (n, 1), 0)
        t = ((iota_col // d) & 1) == 1
        vm = jnp.roll(v, -d, axis=0)
        vp = jnp.roll(v, d, axis=0)
        pv = jnp.where(t, vp, vm)
        im = jnp.roll(i, -d, axis=0)
        ip = jnp.roll(i, d, axis=0)
        pi = jnp.where(t, ip, im)
        r = (v > pv) | ((v == pv) & (i < pi))
        if final_desc:
            keep_first = ~t
        else:
            keep_first = desc ^ t
        sel = r == keep_first
        v = jnp.where(sel, v, pv)
        i = jnp.where(sel, i, pi)
    return v, i


def _sort_chunk(v, i, chunk_desc):
    iota_c = lax.broadcasted_iota(jnp.int32, (512, 1), 0)
    ones = jnp.ones_like(iota_c) > 0
    for k in range(1, 10):
        if k <= 8:
            desc = ((iota_c >> (8 - k)) & 1) == 0
        else:
            desc = ones if chunk_desc else ~ones
        for a in range(k - 1, -1, -1):
            v, i = _substep(v, i, desc, 1 << (8 - a), False)
    return v, i


def _merge_chunk(v, i, chunk_desc):
    for a in range(8, -1, -1):
        if chunk_desc:
            v, i = _substep(v, i, None, 1 << (8 - a), True)
        else:
            iota_c = lax.broadcasted_iota(jnp.int32, (512, 1), 0)
            desc = iota_c < 0
            v, i = _substep(v, i, desc, 1 << (8 - a), False)
    return v, i


def _bitonic_topk_idx(v):
    n = v.shape[0]
    i = lax.broadcasted_iota(jnp.int32, (n, R), 0)
    cs = []
    for c in range(4):
        vc = v[c * 512:(c + 1) * 512, :]
        ic = i[c * 512:(c + 1) * 512, :]
        cs.append(_sort_chunk(vc, ic, chunk_desc=(c % 2 == 0)))
    halves = []
    for p in range(2):
        (va, ia), (vb, ib) = cs[2 * p], cs[2 * p + 1]
        r = (va > vb) | ((va == vb) & (ia < ib))
        vw = jnp.where(r, va, vb)
        iw = jnp.where(r, ia, ib)
        halves.append(_merge_chunk(vw, iw, chunk_desc=(p == 0)))
    (va, ia), (vb, ib) = halves
    r = (va > vb) | ((va == vb) & (ia < ib))
    v = jnp.where(r, va, vb)
    i = jnp.where(r, ia, ib)
    v, i = _merge_chunk(v, i, chunk_desc=True)
    rank = lax.broadcasted_iota(jnp.int32, (TOPK, 1), 0)
    rev = jnp.zeros_like(rank)
    for b in range(9):
        rev = rev | (((rank >> b) & 1) << (8 - b))
    col = lax.broadcasted_iota(jnp.int32, (TOPK, TOPK), 1)
    onehot = (col == rev).astype(jnp.float32)
    out = jnp.dot(onehot, i.astype(jnp.float32),
                  preferred_element_type=jnp.float32)
    return (out + 0.5).astype(jnp.int32)


NSTEP = S // R


def _fused_kernel(qT_in_ref, wq_ref, k_ref, wT_ref, out_ref, acc_ref):
    step = pl.program_id(0)
    par = lax.rem(step, 2)

    @pl.when(step < NSTEP)
    def _scores():
        qT = jnp.dot(wq_ref[...], qT_in_ref[...],
                     preferred_element_type=jnp.float32)
        k = k_ref[...]
        wT = wT_ref[...]
        acc = jnp.zeros((S, R), dtype=jnp.float32)
        for h in range(H):
            sh = jnp.dot(k, qT[h * D:(h + 1) * D, :],
                         preferred_element_type=jnp.float32)
            acc = acc + sh * (wT[h:h + 1, :] * SCALE)
        acc_ref[par] = acc

    @pl.when(step > 0)
    def _sort():
        out_ref[...] = _bitonic_topk_idx(acc_ref[1 - par])


def _layernorm_host(v, gamma, beta, eps=1e-5):
    mu = jnp.mean(v, axis=-1, keepdims=True)
    var = jnp.var(v, axis=-1, keepdims=True)
    return (v - mu) / jnp.sqrt(var + eps) * gamma + beta


def kernel(x, q_input, Wq, Wk, gamma, beta, Ww):
    x2 = x.reshape(S, DM)
    qT_in = q_input.reshape(S, QIN).T
    k = _layernorm_host(x2 @ Wk.T, gamma, beta)
    wT = ((x2 @ Ww.T) * WSCALE).T

    idxT = pl.pallas_call(
        _fused_kernel,
        grid=(NSTEP + 1,),
        in_specs=[
            pl.BlockSpec((QIN, R), lambda i: (0, jnp.minimum(i, NSTEP - 1))),
            pl.BlockSpec((H * D, QIN), lambda i: (0, 0)),
            pl.BlockSpec((S, D), lambda i: (0, 0)),
            pl.BlockSpec((H, R), lambda i: (0, jnp.minimum(i, NSTEP - 1))),
        ],
        out_specs=pl.BlockSpec(
            (TOPK, R), lambda i: (0, jnp.maximum(i - 1, 0))),
        out_shape=jax.ShapeDtypeStruct((TOPK, S), jnp.int32),
        scratch_shapes=[pltpu.VMEM((2, S, R), jnp.float32)],
    )(qT_in, Wq, k, wT)

    return idxT.T.reshape(B, S, TOPK)
